# Initial kernel scaffold; baseline (speedup 1.0000x reference)
import jax
import jax.numpy as jnp
from jax.experimental import pallas as pl


def kernel(x_len, index, r_pos):
    raise NotImplementedError("write your pallas kernel here")


# speedup vs baseline: 1.8398x; 1.8398x over previous
import functools

import jax
import jax.numpy as jnp
from jax import lax
from jax.experimental import pallas as pl
from jax.experimental.pallas import tpu as pltpu
from jax.experimental.pallas import tpu_sc as plsc

_D = 1024
_NC = 2
_NS = 16
_NW = _NC * _NS


@functools.lru_cache(maxsize=None)
def _make_gather(n_rows: int):
    assert n_rows % _NW == 0
    b_per_w = n_rows // _NW
    cb = 64
    assert b_per_w % cb == 0
    n_chunks = b_per_w // cb

    mesh = plsc.VectorSubcoreMesh(core_axis_name="c", subcore_axis_name="s")

    @functools.partial(
        pl.kernel,
        mesh=mesh,
        out_type=jax.ShapeDtypeStruct((n_rows, _D), jnp.float32),
        scratch_types=[
            pltpu.VMEM((b_per_w,), jnp.int32),
            pltpu.VMEM((cb, _D), jnp.float32),
            pltpu.SemaphoreType.DMA,
        ],
    )
    def gather(table_hbm, idx_hbm, out_hbm, idx_v, rows_v, sem):
        wid = lax.axis_index("s") * _NC + lax.axis_index("c")
        base = wid * b_per_w
        pltpu.sync_copy(idx_hbm.at[pl.ds(base, b_per_w)], idx_v)
        for i in range(n_chunks):
            pltpu.async_copy(
                table_hbm.at[idx_v.at[pl.ds(i * cb, cb)]], rows_v, sem
            ).wait()
            pltpu.sync_copy(rows_v, out_hbm.at[pl.ds(base + i * cb, cb)])

    return gather


def kernel(x_len, index, r_pos):
    b, l, _ = index.shape
    table = jnp.reshape(r_pos, (r_pos.shape[1], _D))
    idx = jnp.reshape(index, (b * l,)).astype(jnp.int32)
    out = _make_gather(b * l)(table, idx)
    return jnp.reshape(out, (b, l, _D))


# speedup vs baseline: 1.9510x; 1.0604x over previous
import functools

import jax
import jax.numpy as jnp
from jax import lax
from jax.experimental import pallas as pl
from jax.experimental.pallas import tpu as pltpu
from jax.experimental.pallas import tpu_sc as plsc

_D = 1024
_NC = 2
_NS = 16
_NW = _NC * _NS


@functools.lru_cache(maxsize=None)
def _make_gather(n_rows: int):
    assert n_rows % _NW == 0
    b_per_w = n_rows // _NW
    cb = 32
    assert b_per_w % cb == 0
    n_chunks = b_per_w // cb

    mesh = plsc.VectorSubcoreMesh(core_axis_name="c", subcore_axis_name="s")

    @functools.partial(
        pl.kernel,
        mesh=mesh,
        out_type=jax.ShapeDtypeStruct((n_rows, _D), jnp.float32),
        scratch_types=[
            pltpu.VMEM((b_per_w,), jnp.int32),
            pltpu.VMEM((cb, _D), jnp.float32),
            pltpu.VMEM((cb, _D), jnp.float32),
            pltpu.SemaphoreType.DMA,
            pltpu.SemaphoreType.DMA,
            pltpu.SemaphoreType.DMA,
            pltpu.SemaphoreType.DMA,
        ],
    )
    def gather(table_hbm, idx_hbm, out_hbm, idx_v, rows0, rows1,
               gsem0, gsem1, wsem0, wsem1):
        wid = lax.axis_index("s") * _NC + lax.axis_index("c")
        base = wid * b_per_w
        pltpu.sync_copy(idx_hbm.at[pl.ds(base, b_per_w)], idx_v)
        bufs = (rows0, rows1)
        gsems = (gsem0, gsem1)
        wsems = (wsem0, wsem1)
        gathers = [None, None]
        writes = [None, None]
        gathers[0] = pltpu.async_copy(
            table_hbm.at[idx_v.at[pl.ds(0, cb)]], bufs[0], gsems[0]
        )
        for i in range(n_chunks):
            p = i % 2
            q = (i + 1) % 2
            if i + 1 < n_chunks:
                if writes[q] is not None:
                    writes[q].wait()
                gathers[q] = pltpu.async_copy(
                    table_hbm.at[idx_v.at[pl.ds((i + 1) * cb, cb)]],
                    bufs[q], gsems[q],
                )
            gathers[p].wait()
            writes[p] = pltpu.async_copy(
                bufs[p], out_hbm.at[pl.ds(base + i * cb, cb)], wsems[p]
            )
        writes[(n_chunks - 1) % 2].wait()
        if n_chunks > 1:
            writes[n_chunks % 2].wait()

    return gather


def kernel(x_len, index, r_pos):
    b, l, _ = index.shape
    table = jnp.reshape(r_pos, (r_pos.shape[1], _D))
    idx = jnp.reshape(index, (b * l,)).astype(jnp.int32)
    out = _make_gather(b * l)(table, idx)
    return jnp.reshape(out, (b, l, _D))


# speedup vs baseline: 1.9671x; 1.0083x over previous
import functools

import jax
import jax.numpy as jnp
from jax import lax
from jax.experimental import pallas as pl
from jax.experimental.pallas import tpu as pltpu
from jax.experimental.pallas import tpu_sc as plsc

_D = 1024
_NC = 2
_NS = 16
_NW = _NC * _NS


@functools.lru_cache(maxsize=None)
def _make_gather(n_rows: int):
    assert n_rows % _NW == 0
    b_per_w = n_rows // _NW
    cb = 32
    nbuf = 3
    assert b_per_w % cb == 0
    n_chunks = b_per_w // cb

    mesh = plsc.VectorSubcoreMesh(core_axis_name="c", subcore_axis_name="s")

    @functools.partial(
        pl.kernel,
        mesh=mesh,
        out_type=jax.ShapeDtypeStruct((n_rows, _D), jnp.float32),
        scratch_types=[
            pltpu.VMEM((b_per_w,), jnp.int32),
        ]
        + [pltpu.VMEM((cb, _D), jnp.float32) for _ in range(nbuf)]
        + [pltpu.SemaphoreType.DMA for _ in range(2 * nbuf)],
    )
    def gather(table_hbm, idx_hbm, out_hbm, idx_v, *scratch):
        bufs = scratch[:nbuf]
        gsems = scratch[nbuf:2 * nbuf]
        wsems = scratch[2 * nbuf:]
        wid = lax.axis_index("s") * _NC + lax.axis_index("c")
        base = wid * b_per_w
        pltpu.sync_copy(idx_hbm.at[pl.ds(base, b_per_w)], idx_v)
        gathers = [None] * nbuf
        writes = [None] * nbuf
        for j in range(min(nbuf - 1, n_chunks)):
            gathers[j] = pltpu.async_copy(
                table_hbm.at[idx_v.at[pl.ds(j * cb, cb)]], bufs[j], gsems[j]
            )
        for i in range(n_chunks):
            p = i % nbuf
            nx = i + nbuf - 1
            if nx < n_chunks:
                q = nx % nbuf
                if writes[q] is not None:
                    writes[q].wait()
                    writes[q] = None
                gathers[q] = pltpu.async_copy(
                    table_hbm.at[idx_v.at[pl.ds(nx * cb, cb)]],
                    bufs[q], gsems[q],
                )
            gathers[p].wait()
            writes[p] = pltpu.async_copy(
                bufs[p], out_hbm.at[pl.ds(base + i * cb, cb)], wsems[p]
            )
        for w in writes:
            if w is not None:
                w.wait()

    return gather


def kernel(x_len, index, r_pos):
    b, l, _ = index.shape
    table = jnp.reshape(r_pos, (r_pos.shape[1], _D))
    idx = jnp.reshape(index, (b * l,)).astype(jnp.int32)
    out = _make_gather(b * l)(table, idx)
    return jnp.reshape(out, (b, l, _D))
